# Initial kernel scaffold; baseline (speedup 1.0000x reference)
import functools

import jax
import jax.numpy as jnp
from jax import lax
from jax.experimental import pallas as pl
from jax.experimental.pallas import tpu as pltpu
from jax.experimental.pallas import tpu_sc as plsc

N = 10000
E = 320000
D = 128
H = 128
L = 3

NC = 2
NS = 16
NW = NC * NS

EW = E // NW
CH = 80
NCHUNK = EW // CH
RPS = N // NS

_sc_mesh = plsc.VectorSubcoreMesh(core_axis_name="c", subcore_axis_name="s")


@functools.partial(
    pl.kernel,
    out_type=jax.ShapeDtypeStruct((NC, N, D), jnp.float32),
    mesh=_sc_mesh,
    scratch_types=[
        pltpu.VMEM((NCHUNK, CH), jnp.int32),
        pltpu.VMEM((NCHUNK, CH), jnp.int32),
        pltpu.VMEM((CH, D), jnp.float32),
        pltpu.VMEM_SHARED((N, D), jnp.float32),
        pltpu.SemaphoreType.DMA,
    ],
)
def _sc_aggregate(h_hbm, src_hbm, dst_hbm, zeros_hbm, out_hbm,
                  src_v, dst_v, rows_v, agg_sh, sem):
    cid = lax.axis_index("c")
    sid = lax.axis_index("s")
    wid = cid * NS + sid
    r0 = sid * RPS

    @pl.when(cid == 0)
    def _():
        pltpu.sync_copy(h_hbm.at[pl.ds(r0, RPS)], agg_sh.at[pl.ds(r0, RPS)])

    @pl.when(cid != 0)
    def _():
        pltpu.sync_copy(zeros_hbm, agg_sh.at[pl.ds(r0, RPS)])

    pltpu.sync_copy(src_hbm.at[wid], src_v)
    pltpu.sync_copy(dst_hbm.at[wid], dst_v)

    plsc.subcore_barrier()

    def body(c, carry):
        pltpu.async_copy(h_hbm.at[src_v.at[c]], rows_v, sem).wait()
        pltpu.sync_copy(rows_v, agg_sh.at[dst_v.at[c]], add=True)
        return carry

    lax.fori_loop(0, NCHUNK, body, 0)

    plsc.subcore_barrier()

    pltpu.sync_copy(agg_sh.at[pl.ds(r0, RPS)],
                    out_hbm.at[cid].at[pl.ds(r0, RPS)])


BN = 1000


def _mlp_body(r0_ref, r1_ref, w1_ref, b1_ref, w2_ref, b2_ref,
              h_ref, ro_ref):
    rst = r0_ref[...] + r1_ref[...]
    t = jnp.maximum(
        jnp.dot(rst, w1_ref[...], preferred_element_type=jnp.float32)
        + b1_ref[...][None, :], 0.0)
    o = (jnp.dot(t, w2_ref[...], preferred_element_type=jnp.float32)
         + b2_ref[...][None, :])
    h_ref[...] = o

    @pl.when(pl.program_id(0) == 0)
    def _():
        ro_ref[...] = jnp.zeros_like(ro_ref)

    ro_ref[...] += jnp.sum(o, axis=0, keepdims=True)


def _mlp_layer(r0, r1, W1, b1, W2, b2):
    grid = (N // BN,)
    return pl.pallas_call(
        _mlp_body,
        grid=grid,
        in_specs=[
            pl.BlockSpec((BN, D), lambda i: (i, 0)),
            pl.BlockSpec((BN, D), lambda i: (i, 0)),
            pl.BlockSpec((D, H), lambda i: (0, 0)),
            pl.BlockSpec((H,), lambda i: (0,)),
            pl.BlockSpec((H, H), lambda i: (0, 0)),
            pl.BlockSpec((H,), lambda i: (0,)),
        ],
        out_specs=[
            pl.BlockSpec((BN, H), lambda i: (i, 0)),
            pl.BlockSpec((1, H), lambda i: (0, 0)),
        ],
        out_shape=[
            jax.ShapeDtypeStruct((N, H), jnp.float32),
            jax.ShapeDtypeStruct((1, H), jnp.float32),
        ],
    )(r0, r1, W1, b1, W2, b2)


def _head_body(ro_ref, wr_ref, br_ref, y_ref):
    y_ref[0, 0] = jnp.sum(ro_ref[...] * wr_ref[...]) + br_ref[0]


def _head(ro_all, Wr_r, br):
    return pl.pallas_call(
        _head_body,
        in_specs=[
            pl.BlockSpec((L, H), lambda: (0, 0)),
            pl.BlockSpec((L, H), lambda: (0, 0)),
            pl.BlockSpec(memory_space=pltpu.SMEM),
        ],
        out_specs=pl.BlockSpec((1, 1), lambda: (0, 0)),
        out_shape=jax.ShapeDtypeStruct((1, 1), jnp.float32),
    )(ro_all, Wr_r, br)


def kernel(h, edge_index, W1_0, b1_0, W2_0, b2_0, W1_1, b1_1, W2_1, b2_1,
           W1_2, b1_2, W2_2, b2_2, Wr, br):
    src = edge_index[0].reshape(NW, NCHUNK, CH)
    dst = edge_index[1].reshape(NW, NCHUNK, CH)
    zeros_init = jnp.zeros((RPS, D), jnp.float32)
    params = [(W1_0, b1_0, W2_0, b2_0), (W1_1, b1_1, W2_1, b2_1),
              (W1_2, b1_2, W2_2, b2_2)]

    ros = []
    for (W1, b1, W2, b2) in params:
        parts = _sc_aggregate(h, src, dst, zeros_init)
        h, ro = _mlp_layer(parts[0], parts[1], W1, b1, W2, b2)
        ros.append(ro)

    ro_all = jnp.concatenate(ros, axis=0)
    Wr_r = Wr.reshape(L, H)
    return _head(ro_all, Wr_r, br)


# speedup vs baseline: 6.3353x; 6.3353x over previous
import functools

import jax
import jax.numpy as jnp
from jax import lax
from jax.experimental import pallas as pl
from jax.experimental.pallas import tpu as pltpu
from jax.experimental.pallas import tpu_sc as plsc

N = 10000
E = 320000
D = 128
H = 128
L = 3

NC = 2
NS = 16
NW = NC * NS

EW = E // NW
CH = 80
NCHUNK = EW // CH
RPS = 632
RPS_LAST = N - (NS - 1) * RPS

@functools.lru_cache(maxsize=None)
def _build_sc_aggregate():
    mesh = plsc.VectorSubcoreMesh(core_axis_name="c", subcore_axis_name="s")

    @functools.partial(
        pl.kernel,
        out_type=jax.ShapeDtypeStruct((NC, N, D), jnp.float32),
        mesh=mesh,
        scratch_types=[
            pltpu.VMEM((NCHUNK, CH), jnp.int32),
            pltpu.VMEM((NCHUNK, CH), jnp.int32),
            pltpu.VMEM((CH, D), jnp.float32),
            pltpu.VMEM_SHARED((N, D), jnp.float32),
            pltpu.SemaphoreType.DMA,
        ],
    )
    def _sc_aggregate(h_hbm, src_hbm, dst_hbm, zeros_hbm, out_hbm,
                      src_v, dst_v, rows_v, agg_sh, sem):
        cid = lax.axis_index("c")
        sid = lax.axis_index("s")
        wid = cid * NS + sid
        r0 = sid * RPS
        last = sid == NS - 1

        @pl.when((cid == 0) & ~last)
        def _():
            pltpu.sync_copy(h_hbm.at[pl.ds(r0, RPS)],
                            agg_sh.at[pl.ds(r0, RPS)])

        @pl.when((cid == 0) & last)
        def _():
            pltpu.sync_copy(h_hbm.at[pl.ds(r0, RPS_LAST)],
                            agg_sh.at[pl.ds(r0, RPS_LAST)])

        @pl.when((cid != 0) & ~last)
        def _():
            pltpu.sync_copy(zeros_hbm, agg_sh.at[pl.ds(r0, RPS)])

        @pl.when((cid != 0) & last)
        def _():
            pltpu.sync_copy(zeros_hbm.at[pl.ds(0, RPS_LAST)],
                            agg_sh.at[pl.ds(r0, RPS_LAST)])

        pltpu.sync_copy(src_hbm.at[wid], src_v)
        pltpu.sync_copy(dst_hbm.at[wid], dst_v)

        plsc.subcore_barrier()

        def body(c, carry):
            pltpu.async_copy(h_hbm.at[src_v.at[c]], rows_v, sem).wait()
            pltpu.sync_copy(rows_v, agg_sh.at[dst_v.at[c]], add=True)
            return carry

        lax.fori_loop(0, NCHUNK, body, 0)

        plsc.subcore_barrier()

        @pl.when(~last)
        def _():
            pltpu.sync_copy(agg_sh.at[pl.ds(r0, RPS)],
                            out_hbm.at[cid].at[pl.ds(r0, RPS)])

        @pl.when(last)
        def _():
            pltpu.sync_copy(agg_sh.at[pl.ds(r0, RPS_LAST)],
                            out_hbm.at[cid].at[pl.ds(r0, RPS_LAST)])

    return _sc_aggregate


BN = 1000


def _mlp_body(r0_ref, r1_ref, w1_ref, b1_ref, w2_ref, b2_ref,
              h_ref, ro_ref):
    rst = r0_ref[...] + r1_ref[...]
    t = jnp.maximum(
        jnp.dot(rst, w1_ref[...], preferred_element_type=jnp.float32)
        + b1_ref[...][None, :], 0.0)
    o = (jnp.dot(t, w2_ref[...], preferred_element_type=jnp.float32)
         + b2_ref[...][None, :])
    h_ref[...] = o

    @pl.when(pl.program_id(0) == 0)
    def _():
        ro_ref[...] = jnp.zeros_like(ro_ref)

    ro_ref[...] += jnp.sum(o, axis=0, keepdims=True)


def _mlp_layer(r0, r1, W1, b1, W2, b2):
    grid = (N // BN,)
    return pl.pallas_call(
        _mlp_body,
        grid=grid,
        in_specs=[
            pl.BlockSpec((BN, D), lambda i: (i, 0)),
            pl.BlockSpec((BN, D), lambda i: (i, 0)),
            pl.BlockSpec((D, H), lambda i: (0, 0)),
            pl.BlockSpec((H,), lambda i: (0,)),
            pl.BlockSpec((H, H), lambda i: (0, 0)),
            pl.BlockSpec((H,), lambda i: (0,)),
        ],
        out_specs=[
            pl.BlockSpec((BN, H), lambda i: (i, 0)),
            pl.BlockSpec((1, H), lambda i: (0, 0)),
        ],
        out_shape=[
            jax.ShapeDtypeStruct((N, H), jnp.float32),
            jax.ShapeDtypeStruct((1, H), jnp.float32),
        ],
    )(r0, r1, W1, b1, W2, b2)


def _head_body(ro_ref, wr_ref, br_ref, y_ref):
    y_ref[...] = (jnp.sum(ro_ref[...] * wr_ref[...]) + br_ref[0])[None, None]


def _head(ro_all, Wr_r, br):
    return pl.pallas_call(
        _head_body,
        in_specs=[
            pl.BlockSpec((L, H), lambda: (0, 0)),
            pl.BlockSpec((L, H), lambda: (0, 0)),
            pl.BlockSpec(memory_space=pltpu.SMEM),
        ],
        out_specs=pl.BlockSpec((1, 1), lambda: (0, 0)),
        out_shape=jax.ShapeDtypeStruct((1, 1), jnp.float32),
    )(ro_all, Wr_r, br)


def kernel(h, edge_index, W1_0, b1_0, W2_0, b2_0, W1_1, b1_1, W2_1, b2_1,
           W1_2, b1_2, W2_2, b2_2, Wr, br):
    src = edge_index[0].reshape(NW, NCHUNK, CH)
    dst = edge_index[1].reshape(NW, NCHUNK, CH)
    zeros_init = jnp.zeros((RPS, D), jnp.float32)
    params = [(W1_0, b1_0, W2_0, b2_0), (W1_1, b1_1, W2_1, b2_1),
              (W1_2, b1_2, W2_2, b2_2)]

    ros = []
    for (W1, b1, W2, b2) in params:
        parts = _build_sc_aggregate()(h, src, dst, zeros_init)
        h, ro = _mlp_layer(parts[0], parts[1], W1, b1, W2, b2)
        ros.append(ro)

    ro_all = jnp.concatenate(ros, axis=0)
    Wr_r = Wr.reshape(L, H)
    return _head(ro_all, Wr_r, br)
